# baseline (speedup 1.0000x reference)
import functools

import jax
import jax.numpy as jnp
from jax import lax
from jax.experimental import pallas as pl
from jax.experimental.pallas import tpu as pltpu
from jax.experimental.pallas import tpu_sc as plsc

N_EMB = 8192
DIM = 256
N_TOK = 4096
COMMITMENT_COST = 0.25
TM = 256


def _dist_body(x_ref, w_ref, sx_ref, sw_ref, idx_ref, enc_ref):
    i = pl.program_id(0)
    x = x_ref[...]
    w = w_ref[...]
    s2 = lax.dot_general(x * (-2.0), w, (((1,), (1,)), ((), ())),
                         preferred_element_type=jnp.float32)
    sx = sx_ref[...]
    LW = 128
    RB = 64
    ik0r = lax.broadcasted_iota(jnp.int32, (RB, LW), 1)
    ik0 = lax.broadcasted_iota(jnp.int32, (TM, LW), 1)
    idx_parts = []
    for r in range(TM // RB):
        rs = slice(r * RB, (r + 1) * RB)
        sxr = sx[rs]
        m = mi = None
        for k in range(N_EMB // LW):
            ks = slice(k * LW, (k + 1) * LW)
            dk = (sxr + sw_ref[:, ks]) + s2[rs, ks]
            if k == 0:
                m, mi = dk, jnp.zeros((RB, LW), jnp.int32)
            else:
                lt = dk < m
                m = jnp.where(lt, dk, m)
                mi = jnp.where(lt, jnp.full((RB, LW), k, jnp.int32), mi)
        mv = jnp.min(m, axis=1, keepdims=True)
        idx_parts.append(jnp.min(
            jnp.where(m == mv, mi * LW + ik0r, N_EMB), axis=1, keepdims=True))
    idx = jnp.concatenate(idx_parts, axis=0)
    idx_ref[...] = idx
    for k in range(N_EMB // LW):
        enc_ref[:, k * LW:(k + 1) * LW] = jnp.where(
            ik0 + (k * LW) == idx, 1.0, 0.0).astype(jnp.float32)


def _tc_distance(x, w, sx, sw):
    return pl.pallas_call(
        _dist_body,
        grid=(N_TOK // TM,),
        in_specs=[
            pl.BlockSpec((TM, DIM), lambda i: (i, 0)),
            pl.BlockSpec((N_EMB, DIM), lambda i: (0, 0)),
            pl.BlockSpec((TM, 1), lambda i: (i, 0)),
            pl.BlockSpec((1, N_EMB), lambda i: (0, 0)),
        ],
        out_specs=[
            pl.BlockSpec((TM, 1), lambda i: (i, 0)),
            pl.BlockSpec((TM, N_EMB), lambda i: (i, 0)),
        ],
        out_shape=[
            jax.ShapeDtypeStruct((N_TOK, 1), jnp.int32),
            jax.ShapeDtypeStruct((N_TOK, N_EMB), jnp.float32),
        ],
    )(x, w, sx, sw)


def _sc_quant(w, idx, x):
    info = plsc.get_sparse_core_info()
    nc, ns = info.num_cores, info.num_subcores
    nw = nc * ns
    bpw = N_TOK // nw
    mesh = plsc.VectorSubcoreMesh(core_axis_name="c", subcore_axis_name="s")

    @functools.partial(
        pl.kernel,
        mesh=mesh,
        out_type=[
            jax.ShapeDtypeStruct((N_TOK, DIM), jnp.float32),
            jax.ShapeDtypeStruct((nw, 16), jnp.float32),
            jax.ShapeDtypeStruct((nc, N_EMB), jnp.float32),
        ],
        scratch_types=[
            pltpu.VMEM((bpw,), jnp.int32),
            pltpu.VMEM((bpw, DIM), jnp.float32),
            pltpu.VMEM((bpw, DIM), jnp.float32),
            pltpu.VMEM((16,), jnp.float32),
            pltpu.VMEM((bpw,), jnp.float32),
            pltpu.VMEM((N_EMB,), jnp.float32),
            pltpu.VMEM_SHARED((N_EMB,), jnp.float32),
            pltpu.SemaphoreType.DMA,
        ],
    )
    def k(w_hbm, idx_hbm, x_hbm, qst_hbm, part_hbm, cnt_hbm,
          idx_v, rows_v, x_v, acc_v, ones_v, zero_v, hist_sh, sem):
        core = lax.axis_index("c")
        sid = lax.axis_index("s")
        wid = sid * nc + core
        base = wid * bpw
        pltpu.sync_copy(idx_hbm.at[pl.ds(base, bpw)], idx_v)
        pltpu.sync_copy(x_hbm.at[pl.ds(base, bpw)], x_v)
        pltpu.async_copy(w_hbm.at[idx_v], rows_v, sem).wait()

        @pl.when(sid == 0)
        def _():
            def zbody(j, _):
                zero_v[pl.ds(j * 16, 16)] = jnp.zeros((16,), jnp.float32)
                return 0
            lax.fori_loop(0, N_EMB // 16, zbody, 0)
            pltpu.sync_copy(zero_v, hist_sh)

        for c in range(bpw // 16):
            ones_v[pl.ds(c * 16, 16)] = jnp.ones((16,), jnp.float32)
        acc_v[...] = jnp.zeros((16,), jnp.float32)

        def row_body(r, _):
            for c in range(DIM // 16):
                cs = pl.ds(c * 16, 16)
                q16 = rows_v[r, cs]
                x16 = x_v[r, cs]
                d16 = q16 - x16
                rows_v[r, cs] = x16 + d16
                acc_v[...] = acc_v[...] + d16 * d16
            return 0

        lax.fori_loop(0, bpw, row_body, 0)
        pltpu.sync_copy(rows_v, qst_hbm.at[pl.ds(base, bpw)])
        pltpu.sync_copy(acc_v, part_hbm.at[wid])

        plsc.subcore_barrier()
        pltpu.sync_copy(ones_v, hist_sh.at[idx_v], add=True)
        plsc.subcore_barrier()

        @pl.when(sid == 0)
        def _():
            pltpu.sync_copy(hist_sh, cnt_hbm.at[core])

    return k(w, idx, x)


def _fin_body(c2_ref, part_ref, loss_ref, perp_ref):
    m = jnp.sum(part_ref[...]) * (1.0 / (N_TOK * DIM))
    loss_ref[0, 0] = m + COMMITMENT_COST * m
    p = (c2_ref[0:1, :] + c2_ref[1:2, :]) * (1.0 / N_TOK)
    perp_ref[0, 0] = jnp.exp(-jnp.sum(p * jnp.log(p + 1e-10)))


def _tc_finalize(c2, part):
    return pl.pallas_call(
        _fin_body,
        in_specs=[
            pl.BlockSpec(memory_space=pltpu.VMEM),
            pl.BlockSpec(memory_space=pltpu.VMEM),
        ],
        out_specs=[
            pl.BlockSpec(memory_space=pltpu.SMEM),
            pl.BlockSpec(memory_space=pltpu.SMEM),
        ],
        out_shape=[
            jax.ShapeDtypeStruct((1, 1), jnp.float32),
            jax.ShapeDtypeStruct((1, 1), jnp.float32),
        ],
    )(c2, part)


def kernel(inputs, weight):
    x = jnp.transpose(inputs, (0, 2, 3, 4, 1)).reshape(N_TOK, DIM)
    sx = jnp.sum(x ** 2, axis=1, keepdims=True)
    sw = jnp.sum(weight ** 2, axis=1).reshape(1, N_EMB)
    idx2, enc = _tc_distance(x, weight, sx, sw)
    qst, part, c2 = _sc_quant(weight, idx2.reshape(N_TOK), x)
    loss, perp = _tc_finalize(c2, part)
    quantized_out = jnp.transpose(qst.reshape(4, 4, 16, 16, DIM), (0, 4, 1, 2, 3))
    return (loss.reshape(()), quantized_out, perp.reshape(()), enc)


# speedup vs baseline: 1.1931x; 1.1931x over previous
import functools

import jax
import jax.numpy as jnp
from jax import lax
from jax.experimental import pallas as pl
from jax.experimental.pallas import tpu as pltpu
from jax.experimental.pallas import tpu_sc as plsc

N_EMB = 8192
DIM = 256
N_TOK = 4096
COMMITMENT_COST = 0.25
TM = 512


def _dist_body(x_ref, w_ref, sx_ref, sw_ref, idx_ref, cnt_ref, enc_ref):
    i = pl.program_id(0)
    x = x_ref[...]
    w = w_ref[...]
    s2 = lax.dot_general(x * (-2.0), w, (((1,), (1,)), ((), ())),
                         preferred_element_type=jnp.float32)
    sx = sx_ref[...]
    LW = 128
    RB = 64
    ik0r = lax.broadcasted_iota(jnp.int32, (RB, LW), 1)
    ik0 = lax.broadcasted_iota(jnp.int32, (TM, LW), 1)
    idx_parts = []
    for r in range(TM // RB):
        rs = slice(r * RB, (r + 1) * RB)
        sxr = sx[rs]
        m = mi = None
        for k in range(N_EMB // LW):
            ks = slice(k * LW, (k + 1) * LW)
            dk = (sxr + sw_ref[:, ks]) + s2[rs, ks]
            if k == 0:
                m, mi = dk, jnp.zeros((RB, LW), jnp.int32)
            else:
                lt = dk < m
                m = jnp.where(lt, dk, m)
                mi = jnp.where(lt, jnp.full((RB, LW), k, jnp.int32), mi)
        mv = jnp.min(m, axis=1, keepdims=True)
        idx_parts.append(jnp.min(
            jnp.where(m == mv, mi * LW + ik0r, N_EMB), axis=1, keepdims=True))
    idx = jnp.concatenate(idx_parts, axis=0)
    idx_ref[...] = idx
    cs = []
    for k in range(N_EMB // LW):
        ohk = jnp.where(ik0 + (k * LW) == idx, 1.0, 0.0).astype(jnp.float32)
        enc_ref[:, k * LW:(k + 1) * LW] = ohk
        cs.append(jnp.sum(ohk, axis=0, keepdims=True))

    @pl.when(i == 0)
    def _():
        cnt_ref[...] = jnp.zeros_like(cnt_ref)

    cnt_ref[...] += jnp.concatenate(cs, axis=1)


def _tc_distance(x, w, sx, sw):
    return pl.pallas_call(
        _dist_body,
        grid=(N_TOK // TM,),
        in_specs=[
            pl.BlockSpec((TM, DIM), lambda i: (i, 0)),
            pl.BlockSpec((N_EMB, DIM), lambda i: (0, 0)),
            pl.BlockSpec((TM, 1), lambda i: (i, 0)),
            pl.BlockSpec((1, N_EMB), lambda i: (0, 0)),
        ],
        out_specs=[
            pl.BlockSpec((TM, 1), lambda i: (i, 0)),
            pl.BlockSpec((1, N_EMB), lambda i: (0, 0)),
            pl.BlockSpec((TM, N_EMB), lambda i: (i, 0)),
        ],
        out_shape=[
            jax.ShapeDtypeStruct((N_TOK, 1), jnp.int32),
            jax.ShapeDtypeStruct((1, N_EMB), jnp.float32),
            jax.ShapeDtypeStruct((N_TOK, N_EMB), jnp.float32),
        ],
    )(x, w, sx, sw)


def _sc_gather(w, idx):
    info = plsc.get_sparse_core_info()
    nw = info.num_cores * info.num_subcores
    bpw = N_TOK // nw
    mesh = plsc.VectorSubcoreMesh(core_axis_name="c", subcore_axis_name="s")

    @functools.partial(
        pl.kernel,
        mesh=mesh,
        out_type=jax.ShapeDtypeStruct((N_TOK, DIM), jnp.float32),
        scratch_types=[
            pltpu.VMEM((bpw,), jnp.int32),
            pltpu.VMEM((bpw, DIM), jnp.float32),
            pltpu.SemaphoreType.DMA,
        ],
    )
    def k(w_hbm, idx_hbm, out_hbm, idx_v, rows_v, sem):
        wid = lax.axis_index("s") * info.num_cores + lax.axis_index("c")
        base = wid * bpw
        pltpu.sync_copy(idx_hbm.at[pl.ds(base, bpw)], idx_v)
        pltpu.async_copy(w_hbm.at[idx_v], rows_v, sem).wait()
        pltpu.sync_copy(rows_v, out_hbm.at[pl.ds(base, bpw)])

    return k(w, idx)


def _loss_body(x_ref, q_ref, c_ref, qst_ref, loss_ref, perp_ref):
    x = x_ref[...]
    q = q_ref[...]
    d = q - x
    qst_ref[...] = x + d
    m = jnp.sum(d * d) * (1.0 / (N_TOK * DIM))
    loss_ref[0, 0] = m + COMMITMENT_COST * m
    p = c_ref[...] * (1.0 / N_TOK)
    perp_ref[0, 0] = jnp.exp(-jnp.sum(p * jnp.log(p + 1e-10)))


def _tc_loss(x, q, counts):
    return pl.pallas_call(
        _loss_body,
        in_specs=[
            pl.BlockSpec(memory_space=pltpu.VMEM),
            pl.BlockSpec(memory_space=pltpu.VMEM),
            pl.BlockSpec(memory_space=pltpu.VMEM),
        ],
        out_specs=[
            pl.BlockSpec(memory_space=pltpu.VMEM),
            pl.BlockSpec(memory_space=pltpu.SMEM),
            pl.BlockSpec(memory_space=pltpu.SMEM),
        ],
        out_shape=[
            jax.ShapeDtypeStruct((N_TOK, DIM), jnp.float32),
            jax.ShapeDtypeStruct((1, 1), jnp.float32),
            jax.ShapeDtypeStruct((1, 1), jnp.float32),
        ],
    )(x, q, counts)


def kernel(inputs, weight):
    x = jnp.transpose(inputs, (0, 2, 3, 4, 1)).reshape(N_TOK, DIM)
    sx = jnp.sum(x ** 2, axis=1, keepdims=True)
    sw = jnp.sum(weight ** 2, axis=1).reshape(1, N_EMB)
    idx2, counts, enc = _tc_distance(x, weight, sx, sw)
    q = _sc_gather(weight, idx2.reshape(N_TOK))
    qst, loss, perp = _tc_loss(x, q, counts)
    quantized_out = jnp.transpose(qst.reshape(4, 4, 16, 16, DIM), (0, 4, 1, 2, 3))
    return (loss.reshape(()), quantized_out, perp.reshape(()), enc)
